# baseline (speedup 1.0000x reference)
import functools

import jax
import jax.numpy as jnp
from jax import lax
from jax.experimental import pallas as pl
from jax.experimental.pallas import tpu as pltpu
from jax.experimental.pallas import tpu_sc as plsc

D0, D1, D2 = 5, 6, 2
EMB = 16
NCODES = D0 * D1 * D2
NC, NS, LANES = 2, 16, 16
NW = NC * NS
CHUNK = 2000
GROW = 80
NG = CHUNK // GROW


def _body(a0_hbm, a1_hbm, a2_hbm, w0_hbm, w1_hbm, w2_hbm, out_hbm,
          w0_v, w1_v, w2_v, lut_v, lut_sp, b0_v, b1_v, b2_v, code_v, out_v,
          g_sem, *, per_tile):
    cid = lax.axis_index("c")
    sid = lax.axis_index("s")
    wid = sid * NC + cid

    @pl.when(sid == 0)
    def _():
        pltpu.sync_copy(w0_hbm, w0_v)
        pltpu.sync_copy(w1_hbm, w1_v)
        pltpu.sync_copy(w2_hbm, w2_v)
        for i0 in range(D0):
            r0 = w0_v[i0, :]
            for i1 in range(D1):
                r01 = r0 + w1_v[i1, :]
                for i2 in range(D2):
                    lut_v[(i0 * D1 + i1) * D2 + i2, :] = r01 + w2_v[i2, :]
        pltpu.sync_copy(lut_v, lut_sp)
    plsc.subcore_barrier()

    base_w = wid * per_tile
    nchunks = per_tile // CHUNK

    def chunk_body(k, carry):
        base = base_w + k * CHUNK
        pltpu.sync_copy(a0_hbm.at[pl.ds(base, CHUNK)], b0_v)
        pltpu.sync_copy(a1_hbm.at[pl.ds(base, CHUNK)], b1_v)
        pltpu.sync_copy(a2_hbm.at[pl.ds(base, CHUNK)], b2_v)

        def grp_body(j, carry2):
            for s in range(GROW // LANES):
                off = j * GROW + s * LANES
                code = (b0_v[pl.ds(off, LANES)] * D1
                        + b1_v[pl.ds(off, LANES)]) * D2 + b2_v[pl.ds(off, LANES)]
                code_v[j, pl.ds(s * LANES, LANES)] = code
            return carry2

        lax.fori_loop(0, NG, grp_body, 0)

        descs = [
            pltpu.async_copy(lut_sp.at[code_v.at[j]],
                             out_v.at[pl.ds(j * GROW, GROW)], g_sem)
            for j in range(NG)
        ]
        for d in descs:
            d.wait()
        pltpu.sync_copy(out_v, out_hbm.at[pl.ds(base, CHUNK), :])
        return carry

    lax.fori_loop(0, nchunks, chunk_body, 0)


def kernel(edge_attr, W0, W1, W2):
    E = edge_attr.shape[0]
    per_tile = E // NW
    assert per_tile * NW == E and per_tile % CHUNK == 0, E
    a = edge_attr.astype(jnp.int32)
    a0, a1, a2 = a[:, 0], a[:, 1], a[:, 2]
    mesh = plsc.VectorSubcoreMesh(core_axis_name="c", subcore_axis_name="s",
                                  num_cores=NC, num_subcores=NS)
    return pl.kernel(
        functools.partial(_body, per_tile=per_tile),
        out_type=jax.ShapeDtypeStruct((E, EMB), jnp.float32),
        mesh=mesh,
        compiler_params=pltpu.CompilerParams(needs_layout_passes=False,
                                             use_tc_tiling_on_sc=False),
        scratch_types=[
            pltpu.VMEM((D0, EMB), jnp.float32),
            pltpu.VMEM((D1, EMB), jnp.float32),
            pltpu.VMEM((D2, EMB), jnp.float32),
            pltpu.VMEM((NCODES, EMB), jnp.float32),
            pltpu.VMEM_SHARED((NCODES, EMB), jnp.float32),
            pltpu.VMEM((CHUNK,), jnp.int32),
            pltpu.VMEM((CHUNK,), jnp.int32),
            pltpu.VMEM((CHUNK,), jnp.int32),
            pltpu.VMEM((NG, GROW), jnp.int32),
            pltpu.VMEM((CHUNK, EMB), jnp.float32),
            pltpu.SemaphoreType.DMA,
        ],
    )(a0, a1, a2, W0, W1, W2)
